# baseline (speedup 1.0000x reference)
import functools

import jax
import jax.numpy as jnp
from jax import lax
from jax.experimental import pallas as pl
from jax.experimental.pallas import tpu as pltpu
from jax.experimental.pallas import tpu_sc as plsc

_VOCAB = 262144
_D = 1152
_B = 32768
_NC = 2
_NS = 16
_NW = _NC * _NS
_BPW = _B // _NW
_C = 24
_NBUF = 4
_NFULL = _BPW // _C
_TAIL = _BPW - _NFULL * _C
_NCHUNK = _NFULL + 1
_NGRP = 9

_CHUNKS = [(i * _C, _C) for i in range(_NFULL)] + [(_NFULL * _C, _TAIL)]

_mesh = plsc.VectorSubcoreMesh(
    core_axis_name="c", subcore_axis_name="s", num_cores=_NC, num_subcores=_NS
)


@functools.partial(
    pl.kernel,
    out_type=jax.ShapeDtypeStruct((_B, _D), jnp.float32),
    mesh=_mesh,
    scratch_types=[
        pltpu.VMEM((_BPW,), jnp.int32),
        pltpu.VMEM((_NBUF, _C, _D), jnp.float32),
        pltpu.SemaphoreType.DMA((_NBUF,)),
        pltpu.SemaphoreType.DMA((_NBUF,)),
    ],
)
def _gather(idx_hbm, table_hbm, out_hbm, idx_v, rows, gsem, ssem):
    wid = lax.axis_index("s") * _NC + lax.axis_index("c")
    base = wid * _BPW
    pltpu.sync_copy(idx_hbm.at[pl.ds(base, _BPW)], idx_v)

    def gather_desc(i, b, off=None, sz=_C):
        if off is None:
            off = i * _C
        src = table_hbm.at[idx_v.at[pl.ds(off, sz)]]
        return pltpu.make_async_copy(src, rows.at[b].at[pl.ds(0, sz)], gsem.at[b])

    def scatter_desc(i, b, off=None, sz=_C):
        if off is None:
            off = i * _C
        return pltpu.make_async_copy(
            rows.at[b].at[pl.ds(0, sz)],
            out_hbm.at[pl.ds(base + off, sz)],
            ssem.at[b],
        )

    for b in range(_NBUF):
        gather_desc(b, b).start()

    def group(p, carry):
        j = _NBUF * p
        for b in range(_NBUF):
            gather_desc(j + b, b).wait()
            scatter_desc(j + b, b).start()
        for b in range(_NBUF):
            scatter_desc(j + b, b).wait()
            gather_desc(j + _NBUF + b, b).start()
        return carry

    lax.fori_loop(0, _NGRP, group, 0)

    j = _NBUF * _NGRP
    for b in range(_NBUF):
        gather_desc(j + b, b).wait()
        scatter_desc(j + b, b).start()
    for k in range(j + _NBUF, _NCHUNK):
        b = k % _NBUF
        off, sz = _CHUNKS[k]
        scatter_desc(k - _NBUF, b).wait()
        gather_desc(k, b, off, sz).start()
        gather_desc(k, b, off, sz).wait()
        scatter_desc(k, b, off, sz).start()
    for k in range(_NCHUNK - _NBUF, _NCHUNK):
        b = k % _NBUF
        off, sz = _CHUNKS[k]
        scatter_desc(k, b, off, sz).wait()


def kernel(token_ids, table):
    ids = token_ids.reshape(-1).astype(jnp.int32)
    out = _gather(ids, table)
    return out.reshape(token_ids.shape + (table.shape[1],))


# speedup vs baseline: 1.0146x; 1.0146x over previous
import functools

import jax
import jax.numpy as jnp
from jax import lax
from jax.experimental import pallas as pl
from jax.experimental.pallas import tpu as pltpu
from jax.experimental.pallas import tpu_sc as plsc

_VOCAB = 262144
_D = 1152
_B = 32768
_NC = 2
_NS = 16
_NW = _NC * _NS
_BPW = _B // _NW
_C = 32
_NBUF = 3
_NCHUNK = _BPW // _C
_NGRP = 9

_mesh = plsc.VectorSubcoreMesh(
    core_axis_name="c", subcore_axis_name="s", num_cores=_NC, num_subcores=_NS
)


@functools.partial(
    pl.kernel,
    out_type=jax.ShapeDtypeStruct((_B, _D), jnp.float32),
    mesh=_mesh,
    scratch_types=[
        pltpu.VMEM((_BPW,), jnp.int32),
        pltpu.VMEM((_NBUF, _C, _D), jnp.float32),
        pltpu.SemaphoreType.DMA((_NBUF,)),
        pltpu.SemaphoreType.DMA((_NBUF,)),
    ],
)
def _gather(idx_hbm, table_hbm, out_hbm, idx_v, rows, gsem, ssem):
    wid = lax.axis_index("s") * _NC + lax.axis_index("c")
    base = wid * _BPW
    pltpu.sync_copy(idx_hbm.at[pl.ds(base, _BPW)], idx_v)

    def gather_desc(i, b):
        src = table_hbm.at[idx_v.at[pl.ds(i * _C, _C)]]
        return pltpu.make_async_copy(src, rows.at[b], gsem.at[b])

    def scatter_desc(i, b):
        return pltpu.make_async_copy(
            rows.at[b], out_hbm.at[pl.ds(base + i * _C, _C)], ssem.at[b]
        )

    for b in range(_NBUF):
        gather_desc(b, b).start()

    def group(p, carry):
        j = 3 * p
        for b in range(_NBUF):
            gather_desc(j + b, b).wait()
            scatter_desc(j + b, b).start()
        for b in range(_NBUF):
            scatter_desc(j + b, b).wait()
            gather_desc(j + 3 + b, b).start()
        return carry

    lax.fori_loop(0, _NGRP, group, 0)

    j = 3 * _NGRP
    for b in range(_NBUF):
        gather_desc(j + b, b).wait()
        scatter_desc(j + b, b).start()
    for k, b in ((j + 3, 0), (j + 4, 1)):
        scatter_desc(k - 3, b).wait()
        gather_desc(k, b).start()
    for k, b in ((j + 3, 0), (j + 4, 1)):
        gather_desc(k, b).wait()
        scatter_desc(k, b).start()
    scatter_desc(j + 2, 2).wait()
    scatter_desc(j + 3, 0).wait()
    scatter_desc(j + 4, 1).wait()


def kernel(token_ids, table):
    ids = token_ids.reshape(-1).astype(jnp.int32)
    out = _gather(ids, table)
    return out.reshape(token_ids.shape + (table.shape[1],))
